# baseline (speedup 1.0000x reference)
import functools

import jax
import jax.numpy as jnp
from jax import lax
from jax.experimental import pallas as pl
from jax.experimental.pallas import tpu as pltpu
from jax.experimental.pallas import tpu_sc as plsc

N = 10000
D = 256
H = 256
E = 160000
HALF = 128
BN_EPS = 1e-5

NC = 2
NS = 16
CHUNK = 125
EPAD = E
ROWS_2D = EPAD // CHUNK
CHUNKS_PER_TILE = ROWS_2D // NS
NPAD = 10240
TRASH = 10200
ROWS_PER_TILE = NPAD // NS
STAGE = 32
PAGE = 16
NPAGES = CHUNKS_PER_TILE // PAGE


def _sc_aggregate(v_l, v_r, src2d, dst2d):
    mesh = plsc.VectorSubcoreMesh(core_axis_name="c", subcore_axis_name="s")
    hout = jax.ShapeDtypeStruct((NPAD, HALF), jnp.float32)
    ipage = pltpu.VMEM((PAGE, CHUNK), jnp.int32)

    @functools.partial(
        pl.kernel,
        out_type=[hout, hout],
        mesh=mesh,
        scratch_types=[
            pltpu.VMEM_SHARED((NPAD, HALF), jnp.float32),
            pltpu.VMEM((STAGE, HALF), jnp.float32),
            ipage, ipage,
            ipage, ipage,
            pltpu.VMEM((CHUNK, HALF), jnp.float32),
            pltpu.VMEM((CHUNK, HALF), jnp.float32),
            pltpu.SemaphoreType.DMA,
            pltpu.SemaphoreType.DMA,
            pltpu.SemaphoreType.DMA,
            pltpu.SemaphoreType.DMA,
        ],
    )
    def agg(vl_hbm, vr_hbm, src_hbm, dst_hbm, out_l, out_r,
            accum, stage, sp0, sp1, dp0, dp1, rows_a, rows_b,
            sem_a, sem_b, sem_i, sem_z):
        c = lax.axis_index("c")
        s = lax.axis_index("s")
        spages = (sp0, sp1)
        dpages = (dp0, dp1)
        base = s * CHUNKS_PER_TILE

        psl = pl.ds(base, PAGE)
        pltpu.async_copy(src_hbm.at[psl], sp0, sem_i)
        pltpu.async_copy(dst_hbm.at[psl], dp0, sem_i)
        zeros16 = jnp.zeros((16,), jnp.float32)

        def zbody(t, carry):
            r = t // (HALF // 16)
            cc = (t % (HALF // 16)) * 16
            stage[r, pl.ds(cc, 16)] = zeros16
            return carry

        lax.fori_loop(0, STAGE * (HALF // 16), zbody, 0)
        for k in range(ROWS_PER_TILE // STAGE):
            pltpu.async_copy(
                stage, accum.at[pl.ds(s * ROWS_PER_TILE + k * STAGE, STAGE)], sem_z)
        for k in range(ROWS_PER_TILE // STAGE):
            pltpu.make_async_copy(
                stage, accum.at[pl.ds(s * ROWS_PER_TILE + k * STAGE, STAGE)], sem_z).wait()

        pltpu.make_async_copy(src_hbm.at[psl], sp0, sem_i).wait()
        pltpu.make_async_copy(dst_hbm.at[psl], dp0, sem_i).wait()

        plsc.subcore_barrier()

        def main_loop(vh):
            pltpu.async_copy(vh.at[sp0.at[0]], rows_a, sem_a)
            for p in range(NPAGES):
                cs, cd = spages[p % 2], dpages[p % 2]
                ns, nd = spages[(p + 1) % 2], dpages[(p + 1) % 2]
                nsl = pl.ds(base + (p + 1) * PAGE, PAGE)
                if p + 1 < NPAGES:
                    pltpu.async_copy(src_hbm.at[nsl], ns, sem_i)
                    pltpu.async_copy(dst_hbm.at[nsl], nd, sem_i)

                def body(j, carry):
                    b = 2 * j
                    pltpu.async_copy(vh.at[cs.at[b + 1]], rows_b, sem_b)
                    pltpu.make_async_copy(vh.at[cs.at[b]], rows_a, sem_a).wait()
                    pltpu.sync_copy(rows_a, accum.at[cd.at[b]], add=True)

                    @pl.when(j < PAGE // 2 - 1)
                    def _():
                        pltpu.async_copy(vh.at[cs.at[b + 2]], rows_a, sem_a)

                    pltpu.make_async_copy(vh.at[cs.at[b + 1]], rows_b, sem_b).wait()
                    pltpu.sync_copy(rows_b, accum.at[cd.at[b + 1]], add=True)
                    return carry

                lax.fori_loop(0, PAGE // 2, body, 0)

                if p + 1 < NPAGES:
                    pltpu.make_async_copy(src_hbm.at[nsl], ns, sem_i).wait()
                    pltpu.make_async_copy(dst_hbm.at[nsl], nd, sem_i).wait()
                    pltpu.async_copy(vh.at[ns.at[0]], rows_a, sem_a)

        @pl.when(c == 0)
        def _():
            main_loop(vl_hbm)

        @pl.when(c == 1)
        def _():
            main_loop(vr_hbm)

        plsc.subcore_barrier()

        def copy_out(o):
            sl = pl.ds(s * ROWS_PER_TILE, ROWS_PER_TILE)
            pltpu.sync_copy(accum.at[sl], o.at[sl])

        @pl.when(c == 0)
        def _():
            copy_out(out_l)

        @pl.when(c == 1)
        def _():
            copy_out(out_r)

    return agg(v_l, v_r, src2d, dst2d)


TILE = 2000
GRID = N // TILE


def _bn_scale(ssum, sq, g):
    mean = ssum[0:1, :] * (1.0 / N)
    var = sq[0:1, :] * (1.0 / N) - mean * mean
    rstd = lax.rsqrt(var + BN_EPS)
    return mean, rstd * g[...]


def _mlp_body(al, ar, v, eps, w1, b1, g1, bt1, w2, b2, g2, bt2, out,
              h1_s, h2_s, s1, q1, s2, q2):
    i = pl.program_id(0)
    ph = i // GRID
    t = i - ph * GRID
    rows = pl.ds(t * TILE, TILE)

    @pl.when(ph == 0)
    def _():
        x = jnp.concatenate([al[...], ar[...]], axis=1)
        x = x + (1.0 + eps[0, 0]) * v[...]
        h = jnp.dot(x, w1[...], preferred_element_type=jnp.float32) + b1[...]
        h1_s[rows, :] = h

        @pl.when(t == 0)
        def _():
            s1[...] = jnp.zeros_like(s1)
            q1[...] = jnp.zeros_like(q1)

        ones = jnp.ones((8, TILE), jnp.float32)
        s1[...] += jnp.dot(ones, h, preferred_element_type=jnp.float32)
        q1[...] += jnp.dot(ones, h * h, preferred_element_type=jnp.float32)

    @pl.when(ph == 1)
    def _():
        mean, scale = _bn_scale(s1, q1, g1)
        hn = jnp.maximum((h1_s[rows, :] - mean) * scale + bt1[...], 0.0)
        h = jnp.dot(hn, w2[...], preferred_element_type=jnp.float32) + b2[...]
        h2_s[rows, :] = h

        @pl.when(t == 0)
        def _():
            s2[...] = jnp.zeros_like(s2)
            q2[...] = jnp.zeros_like(q2)

        ones = jnp.ones((8, TILE), jnp.float32)
        s2[...] += jnp.dot(ones, h, preferred_element_type=jnp.float32)
        q2[...] += jnp.dot(ones, h * h, preferred_element_type=jnp.float32)

    @pl.when(ph == 2)
    def _():
        mean, scale = _bn_scale(s2, q2, g2)
        out[...] = jnp.maximum((h2_s[rows, :] - mean) * scale + bt2[...], 0.0)


def _row_spec(w):
    return pl.BlockSpec((TILE, w), lambda i: (i, 0))


def _full_spec(shape):
    return pl.BlockSpec(shape, lambda i: (0,) * len(shape))


def kernel(v, edge_index, epsilon, W1, b1, g1, bt1, W2, b2, g2, bt2):
    v_l = v[:, :HALF]
    v_r = v[:, HALF:]
    src2d = edge_index[1].reshape(ROWS_2D, CHUNK)
    dst2d = edge_index[0].reshape(ROWS_2D, CHUNK)

    agg_l, agg_r = _sc_aggregate(v_l, v_r, src2d, dst2d)

    b1_ = b1.reshape(1, H)
    g1_ = g1.reshape(1, H)
    bt1_ = bt1.reshape(1, H)
    b2_ = b2.reshape(1, H)
    g2_ = g2.reshape(1, H)
    bt2_ = bt2.reshape(1, H)

    def in_rows(w):
        return pl.BlockSpec(
            (TILE, w), lambda i: (jnp.where(i < GRID, i, 0), 0))

    out_rows = pl.BlockSpec(
        (TILE, H), lambda i: (jnp.where(i >= 2 * GRID, i - 2 * GRID, 0), 0))

    out = pl.pallas_call(
        _mlp_body,
        grid=(3 * GRID,),
        in_specs=[
            in_rows(HALF), in_rows(HALF), in_rows(D),
            pl.BlockSpec(memory_space=pltpu.SMEM),
            _full_spec((D, H)), _full_spec((1, H)),
            _full_spec((1, H)), _full_spec((1, H)),
            _full_spec((H, H)), _full_spec((1, H)),
            _full_spec((1, H)), _full_spec((1, H)),
        ],
        out_specs=out_rows,
        out_shape=jax.ShapeDtypeStruct((N, H), jnp.float32),
        scratch_shapes=[
            pltpu.VMEM((N, H), jnp.float32),
            pltpu.VMEM((N, H), jnp.float32),
            pltpu.VMEM((8, H), jnp.float32),
            pltpu.VMEM((8, H), jnp.float32),
            pltpu.VMEM((8, H), jnp.float32),
            pltpu.VMEM((8, H), jnp.float32),
        ],
    )(agg_l, agg_r, v, epsilon, W1, b1_, g1_, bt1_, W2, b2_, g2_, bt2_)

    return out


# speedup vs baseline: 1.1644x; 1.1644x over previous
import functools

import jax
import jax.numpy as jnp
from jax import lax
from jax.experimental import pallas as pl
from jax.experimental.pallas import tpu as pltpu
from jax.experimental.pallas import tpu_sc as plsc

N = 10000
D = 256
H = 256
E = 160000
HALF = 128
BN_EPS = 1e-5

NC = 2
NS = 16
CHUNK = 125
EPAD = E
ROWS_2D = EPAD // CHUNK
CHUNKS_PER_TILE = ROWS_2D // NS
NPAD = 10240
TRASH = 10200
ROWS_PER_TILE = NPAD // NS
STAGE = 32
PAGE = 16
NPAGES = CHUNKS_PER_TILE // PAGE


def _sc_aggregate(v, src2d, dst2d):
    mesh = plsc.VectorSubcoreMesh(core_axis_name="c", subcore_axis_name="s")
    hout = jax.ShapeDtypeStruct((NPAD, HALF), jnp.float32)
    ipage = pltpu.VMEM((PAGE, CHUNK), jnp.int32)

    @functools.partial(
        pl.kernel,
        out_type=[hout, hout],
        mesh=mesh,
        scratch_types=[
            pltpu.VMEM_SHARED((NPAD, HALF), jnp.float32),
            pltpu.VMEM((STAGE, HALF), jnp.float32),
            ipage, ipage,
            ipage, ipage,
            pltpu.VMEM((CHUNK, HALF), jnp.float32),
            pltpu.VMEM((CHUNK, HALF), jnp.float32),
            pltpu.SemaphoreType.DMA,
            pltpu.SemaphoreType.DMA,
            pltpu.SemaphoreType.DMA,
            pltpu.SemaphoreType.DMA,
        ],
    )
    def agg(v_hbm, src_hbm, dst_hbm, out_l, out_r,
            accum, stage, sp0, sp1, dp0, dp1, rows_a, rows_b,
            sem_a, sem_b, sem_i, sem_z):
        c = lax.axis_index("c")
        s = lax.axis_index("s")
        spages = (sp0, sp1)
        dpages = (dp0, dp1)
        base = s * CHUNKS_PER_TILE

        psl = pl.ds(base, PAGE)
        pltpu.async_copy(src_hbm.at[psl], sp0, sem_i)
        pltpu.async_copy(dst_hbm.at[psl], dp0, sem_i)
        zeros16 = jnp.zeros((16,), jnp.float32)

        def zbody(t, carry):
            r = t // (HALF // 16)
            cc = (t % (HALF // 16)) * 16
            stage[r, pl.ds(cc, 16)] = zeros16
            return carry

        lax.fori_loop(0, STAGE * (HALF // 16), zbody, 0)
        for k in range(ROWS_PER_TILE // STAGE):
            pltpu.async_copy(
                stage, accum.at[pl.ds(s * ROWS_PER_TILE + k * STAGE, STAGE)], sem_z)
        for k in range(ROWS_PER_TILE // STAGE):
            pltpu.make_async_copy(
                stage, accum.at[pl.ds(s * ROWS_PER_TILE + k * STAGE, STAGE)], sem_z).wait()

        pltpu.make_async_copy(src_hbm.at[psl], sp0, sem_i).wait()
        pltpu.make_async_copy(dst_hbm.at[psl], dp0, sem_i).wait()

        plsc.subcore_barrier()

        def main_loop(vh):
            pltpu.async_copy(vh.at[sp0.at[0]], rows_a, sem_a)

            def inner(cs, cd):
                def body(j, carry):
                    b = 2 * j
                    pltpu.async_copy(vh.at[cs.at[b + 1]], rows_b, sem_b)
                    pltpu.make_async_copy(vh.at[cs.at[b]], rows_a, sem_a).wait()
                    pltpu.sync_copy(rows_a, accum.at[cd.at[b]], add=True)

                    @pl.when(j < PAGE // 2 - 1)
                    def _():
                        pltpu.async_copy(vh.at[cs.at[b + 2]], rows_a, sem_a)

                    pltpu.make_async_copy(vh.at[cs.at[b + 1]], rows_b, sem_b).wait()
                    pltpu.sync_copy(rows_b, accum.at[cd.at[b + 1]], add=True)
                    return carry

                lax.fori_loop(0, PAGE // 2, body, 0)

            def pair(k, carry):
                sl1 = pl.ds(base + (2 * k + 1) * PAGE, PAGE)
                pltpu.async_copy(src_hbm.at[sl1], sp1, sem_i)
                pltpu.async_copy(dst_hbm.at[sl1], dp1, sem_i)
                inner(sp0, dp0)
                pltpu.make_async_copy(src_hbm.at[sl1], sp1, sem_i).wait()
                pltpu.make_async_copy(dst_hbm.at[sl1], dp1, sem_i).wait()
                pltpu.async_copy(vh.at[sp1.at[0]], rows_a, sem_a)

                sl2 = pl.ds(base + jnp.minimum(2 * k + 2, NPAGES - 1) * PAGE, PAGE)

                @pl.when(k < NPAGES // 2 - 1)
                def _():
                    pltpu.async_copy(src_hbm.at[sl2], sp0, sem_i)
                    pltpu.async_copy(dst_hbm.at[sl2], dp0, sem_i)

                inner(sp1, dp1)

                @pl.when(k < NPAGES // 2 - 1)
                def _():
                    pltpu.make_async_copy(src_hbm.at[sl2], sp0, sem_i).wait()
                    pltpu.make_async_copy(dst_hbm.at[sl2], dp0, sem_i).wait()
                    pltpu.async_copy(vh.at[sp0.at[0]], rows_a, sem_a)

                return carry

            lax.fori_loop(0, NPAGES // 2, pair, 0)

        @pl.when(c == 0)
        def _():
            main_loop(v_hbm.at[:, pl.ds(0, HALF)])

        @pl.when(c == 1)
        def _():
            main_loop(v_hbm.at[:, pl.ds(HALF, HALF)])

        plsc.subcore_barrier()

        def copy_out(o):
            sl = pl.ds(s * ROWS_PER_TILE, ROWS_PER_TILE)
            pltpu.sync_copy(accum.at[sl], o.at[sl])

        @pl.when(c == 0)
        def _():
            copy_out(out_l)

        @pl.when(c == 1)
        def _():
            copy_out(out_r)

    return agg(v, src2d, dst2d)


TILE = 2000
GRID = N // TILE


def _bn_scale(ssum, sq, g):
    mean = ssum[0:1, :] * (1.0 / N)
    var = sq[0:1, :] * (1.0 / N) - mean * mean
    rstd = lax.rsqrt(var + BN_EPS)
    return mean, rstd * g[...]


def _mlp_body(al, ar, v, eps, w1, b1, g1, bt1, w2, b2, g2, bt2, out,
              h1_s, h2_s, s1, q1, s2, q2):
    i = pl.program_id(0)
    ph = i // GRID
    t = i - ph * GRID
    rows = pl.ds(t * TILE, TILE)

    @pl.when(ph == 0)
    def _():
        x = jnp.concatenate([al[...], ar[...]], axis=1)
        x = x + (1.0 + eps[0, 0]) * v[...]
        h = jnp.dot(x, w1[...], preferred_element_type=jnp.float32) + b1[...]
        h1_s[rows, :] = h

        @pl.when(t == 0)
        def _():
            s1[...] = jnp.zeros_like(s1)
            q1[...] = jnp.zeros_like(q1)

        s1[0:1, :] += jnp.sum(h, axis=0, keepdims=True)
        q1[0:1, :] += jnp.sum(h * h, axis=0, keepdims=True)

    @pl.when(ph == 1)
    def _():
        mean, scale = _bn_scale(s1, q1, g1)
        hn = jnp.maximum((h1_s[rows, :] - mean) * scale + bt1[...], 0.0)
        h = jnp.dot(hn, w2[...], preferred_element_type=jnp.float32) + b2[...]
        h2_s[rows, :] = h

        @pl.when(t == 0)
        def _():
            s2[...] = jnp.zeros_like(s2)
            q2[...] = jnp.zeros_like(q2)

        s2[0:1, :] += jnp.sum(h, axis=0, keepdims=True)
        q2[0:1, :] += jnp.sum(h * h, axis=0, keepdims=True)

    @pl.when(ph == 2)
    def _():
        mean, scale = _bn_scale(s2, q2, g2)
        out[...] = jnp.maximum((h2_s[rows, :] - mean) * scale + bt2[...], 0.0)


def _row_spec(w):
    return pl.BlockSpec((TILE, w), lambda i: (i, 0))


def _full_spec(shape):
    return pl.BlockSpec(shape, lambda i: (0,) * len(shape))


def kernel(v, edge_index, epsilon, W1, b1, g1, bt1, W2, b2, g2, bt2):
    src2d = edge_index[1].reshape(ROWS_2D, CHUNK)
    dst2d = edge_index[0].reshape(ROWS_2D, CHUNK)

    agg_l, agg_r = _sc_aggregate(v, src2d, dst2d)

    b1_ = b1.reshape(1, H)
    g1_ = g1.reshape(1, H)
    bt1_ = bt1.reshape(1, H)
    b2_ = b2.reshape(1, H)
    g2_ = g2.reshape(1, H)
    bt2_ = bt2.reshape(1, H)

    def in_rows(w):
        return pl.BlockSpec(
            (TILE, w), lambda i: (jnp.where(i < GRID, i, 0), 0))

    out_rows = pl.BlockSpec(
        (TILE, H), lambda i: (jnp.where(i >= 2 * GRID, i - 2 * GRID, 0), 0))

    out = pl.pallas_call(
        _mlp_body,
        grid=(3 * GRID,),
        in_specs=[
            in_rows(HALF), in_rows(HALF), in_rows(D),
            pl.BlockSpec(memory_space=pltpu.SMEM),
            _full_spec((D, H)), _full_spec((1, H)),
            _full_spec((1, H)), _full_spec((1, H)),
            _full_spec((H, H)), _full_spec((1, H)),
            _full_spec((1, H)), _full_spec((1, H)),
        ],
        out_specs=out_rows,
        out_shape=jax.ShapeDtypeStruct((N, H), jnp.float32),
        scratch_shapes=[
            pltpu.VMEM((N, H), jnp.float32),
            pltpu.VMEM((N, H), jnp.float32),
            pltpu.VMEM((8, H), jnp.float32),
            pltpu.VMEM((8, H), jnp.float32),
            pltpu.VMEM((8, H), jnp.float32),
            pltpu.VMEM((8, H), jnp.float32),
        ],
    )(agg_l, agg_r, v, epsilon, W1, b1_, g1_, bt1_, W2, b2_, g2_, bt2_)

    return out
